# baseline (speedup 1.0000x reference)
import functools

import jax
import jax.numpy as jnp
from jax.experimental import pallas as pl
from jax.experimental.pallas import tpu as pltpu

_PAIR = 256


def _round_up(x, m):
    return (x + m - 1) // m * m


def _ffn_block_kernel(h_ref, fnw_ref, w13_ref, w2_ref, anw_ref,
                      o_ref, x_ref, *, eps, inv_dim, tail_pairs):
    k = pl.program_id(1)
    nk = pl.num_programs(1)

    @pl.when(k == 0)
    def _init():
        h = h_ref[...]
        ms = jnp.sum(h * h, axis=-1, keepdims=True) * inv_dim
        x_ref[...] = (h * jax.lax.rsqrt(ms + eps) * fnw_ref[...]).astype(x_ref.dtype)
        o_ref[...] = h

    x = x_ref[...]

    def gate(hh, p):
        return jax.nn.silu(hh[:, 2 * p * _PAIR:(2 * p + 1) * _PAIR]) \
            * hh[:, (2 * p + 1) * _PAIR:(2 * p + 2) * _PAIR]

    @pl.when(k < nk - 1)
    def _full_step():
        hh = jnp.dot(x, w13_ref[...], preferred_element_type=jnp.float32)
        gated = jnp.concatenate([gate(hh, 0), gate(hh, 1)], axis=1)
        o_ref[...] += jnp.dot(gated.astype(w2_ref.dtype), w2_ref[...],
                              preferred_element_type=jnp.float32)

    @pl.when(k == nk - 1)
    def _tail_step():
        hh = jnp.dot(x, w13_ref[:, :2 * tail_pairs * _PAIR],
                     preferred_element_type=jnp.float32)
        gated = gate(hh, 0) if tail_pairs == 1 else \
            jnp.concatenate([gate(hh, 0), gate(hh, 1)], axis=1)
        o_ref[...] += jnp.dot(gated.astype(w2_ref.dtype),
                              w2_ref[:tail_pairs * _PAIR, :],
                              preferred_element_type=jnp.float32)

        y = o_ref[...]
        ms2 = jnp.sum(y * y, axis=-1, keepdims=True) * inv_dim
        o_ref[...] = y * jax.lax.rsqrt(ms2 + eps) * anw_ref[...]


def kernel(h, ffn_nw, w13, w2, attn_nw, *, eps=1e-6):
    B, S, dim = h.shape
    dim_p = ffn_nw.shape[1]
    tokens = B * S

    npairs = w13.shape[1] // (2 * _PAIR)
    nk = (npairs + 1) // 2
    tail_pairs = npairs - 2 * (nk - 1)
    tb = 4 * _PAIR

    tm = 512
    while tokens % tm and tm > 8:
        tm //= 2
    tokens_p = _round_up(tokens, tm)
    n_tiles = tokens_p // tm

    h2d = h.reshape(tokens, dim)
    if tokens_p != tokens or dim_p != dim:
        h2d = jnp.pad(h2d, ((0, tokens_p - tokens), (0, dim_p - dim)))

    w_bytes = (w13.size + w2.size) * w13.dtype.itemsize
    cost = pl.CostEstimate(
        flops=int(6 * tokens_p * dim_p * npairs * _PAIR),
        transcendentals=int(tokens_p * npairs * _PAIR + 2 * tokens_p),
        bytes_accessed=int(w_bytes * n_tiles + 2 * tokens_p * dim_p * 4),
    )

    body = functools.partial(_ffn_block_kernel, eps=eps, inv_dim=1.0 / dim,
                             tail_pairs=tail_pairs)

    out = pl.pallas_call(
        body,
        out_shape=jax.ShapeDtypeStruct((tokens_p, dim_p), h.dtype),
        grid=(n_tiles, nk),
        in_specs=[
            pl.BlockSpec((tm, dim_p), lambda i, k: (i, 0),
                         pipeline_mode=pl.Buffered(buffer_count=1)),
            pl.BlockSpec((1, dim_p), lambda i, k: (0, 0)),
            pl.BlockSpec((dim_p, tb), lambda i, k: (0, k)),
            pl.BlockSpec((tb // 2, dim_p), lambda i, k: (k, 0)),
            pl.BlockSpec((1, dim_p), lambda i, k: (0, 0)),
        ],
        out_specs=pl.BlockSpec((tm, dim_p), lambda i, k: (i, 0),
                               pipeline_mode=pl.Buffered(buffer_count=1)),
        scratch_shapes=[pltpu.VMEM((tm, dim_p), w13.dtype)],
        compiler_params=pltpu.CompilerParams(
            dimension_semantics=("parallel", "arbitrary"),
            vmem_limit_bytes=60 * 1024 * 1024,
        ),
        cost_estimate=cost,
    )(h2d, ffn_nw, w13, w2, attn_nw)

    if tokens_p != tokens or dim_p != dim:
        out = out[:tokens, :dim]
    return out.reshape(B, S, dim)
